# Initial kernel scaffold; baseline (speedup 1.0000x reference)
import jax
import jax.numpy as jnp
from jax.experimental import pallas as pl


def kernel(x_inst, x_data, ei_control, ei_input, ei_output, ei_call, params):
    raise NotImplementedError("write your pallas kernel here")


# speedup vs baseline: 6.5837x; 6.5837x over previous
import functools

import jax
import jax.numpy as jnp
from jax import lax
from jax.experimental import pallas as pl
from jax.experimental.pallas import tpu as pltpu
from jax.experimental.pallas import tpu_sc as plsc

H = 4
DH = 32
D = 128
N = 50000
E = 200000

NW = 32
NPASS = 4
R = 392
NPAD = NW * NPASS * R
C = 16


def _proj_body(x_ref, w_ref, b_ref, o_ref):
    o_ref[...] = (
        jnp.dot(x_ref[...], w_ref[...], preferred_element_type=jnp.float32)
        + b_ref[...]
    )


def _project(x, w, b, blk=2000):
    n, d = x.shape
    cols = w.shape[1]
    grid = n // blk
    return pl.pallas_call(
        _proj_body,
        grid=(grid,),
        in_specs=[
            pl.BlockSpec((blk, d), lambda i: (i, 0)),
            pl.BlockSpec((d, cols), lambda i: (0, 0)),
            pl.BlockSpec((1, cols), lambda i: (0, 0)),
        ],
        out_specs=pl.BlockSpec((blk, cols), lambda i: (i, 0)),
        out_shape=jax.ShapeDtypeStruct((n, cols), jnp.float32),
    )(x, w, b.reshape(1, cols))


def _vgat(v, idx):
    dn = lax.GatherDimensionNumbers(
        offset_dims=(), collapsed_slice_dims=(0,), start_index_map=(0,))
    return lax.gather(v, idx[:, None], dn, (1,),
                      mode=lax.GatherScatterMode.PROMISE_IN_BOUNDS)


def _sc_body(q_hbm, k_hbm, v_hbm, si_hbm, di_hbm, starts_hbm,
             num_hbm, den_hbm,
             starts_v, si_v, di_v, qrows, krows, vrows, qc, kc, vc,
             acc_num, acc_den, sem_q, sem_k, sem_v):
    wid = lax.axis_index("s") * 2 + lax.axis_index("c")
    lanes = lax.iota(jnp.int32, 16)

    pltpu.sync_copy(starts_hbm, starts_v)

    def extract(idx):
        vec = starts_v[pl.ds(idx, 16)]
        return vec[0]

    def pass_body(p, carry):
        ridx = p * NW + wid
        range_lo = ridx * R

        zv = jnp.zeros((16,), jnp.float32)

        def zbody(r, c2):
            acc_num[pl.ds(r * 16, 16)] = zv
            return c2
        lax.fori_loop(0, R * D // 16, zbody, 0)

        def zbody2(r, c2):
            acc_den[pl.ds(r * 16, 16)] = zv
            return c2
        lax.fori_loop(0, R, zbody2, 0)

        eb = extract(ridx)
        ee = extract(ridx + 1)
        c0 = eb // 16
        c1 = (ee + 15) // 16

        def chunk_body(ci, c3):
            base = ci * 16
            pltpu.sync_copy(si_hbm.at[pl.ds(base, 16)], si_v)
            pltpu.sync_copy(di_hbm.at[pl.ds(base, 16)], di_v)
            si = si_v[...]
            di = di_v[...]
            eidx = base + lanes
            m = (eidx >= eb) & (eidx < ee)
            si_c = jnp.where(m, si, 0)
            di_g = jnp.where(m, di, 0)
            dloc = jnp.clip(di - range_lo, 0, R - 1)

            cq = pltpu.async_copy(q_hbm.at[di_g], qrows, sem_q)
            ck = pltpu.async_copy(k_hbm.at[si_c], krows, sem_k)
            cv = pltpu.async_copy(v_hbm.at[si_c], vrows, sem_v)
            cq.wait()
            ck.wait()
            cv.wait()

            for src2d, dst1d in ((qrows, qc), (krows, kc), (vrows, vc)):
                for j in range(16):
                    for s in range(8):
                        rowvec = src2d[j, pl.ds(s * 16, 16)]
                        plsc.store_scatter(
                            dst1d, [(s * 16 + lanes) * 16 + j], rowvec)

            def colv(ref1d, col):
                return ref1d[pl.ds(col * 16, 16)]

            exs = []
            for h in range(H):
                a = jnp.zeros((16,), jnp.float32)
                for dd in range(DH):
                    col = h * DH + dd
                    a = a + colv(qc, col) * colv(kc, col)
                exs.append(jnp.where(m, jnp.exp(a), 0.0))

            d_next = _vgat(di, jnp.minimum(lanes + 1, 15))
            is_last = (lanes == 15) | (d_next != di)
            mask_last = m & is_last
            steps = []
            for st in (1, 2, 4, 8):
                idx_st = jnp.maximum(lanes - st, 0)
                steps.append(
                    (idx_st, (lanes >= st) & (_vgat(di, idx_st) == di)))

            def segsum(vals):
                for idx_st, mseg in steps:
                    vals = vals + jnp.where(mseg, _vgat(vals, idx_st), 0.0)
                return vals

            for col in range(D):
                h = col // DH
                vals = exs[h] * colv(vc, col)
                plsc.addupdate_scatter(
                    acc_num, [dloc * D + col], segsum(vals), mask=mask_last)
            for h in range(H):
                plsc.addupdate_scatter(
                    acc_den, [dloc * 16 + h], segsum(exs[h]), mask=mask_last)
            return c3

        lax.fori_loop(c0, c1, chunk_body, 0)

        pltpu.sync_copy(acc_num, num_hbm.at[pl.ds(range_lo * D, R * D)])
        pltpu.sync_copy(acc_den, den_hbm.at[pl.ds(range_lo * 16, R * 16)])
        return carry

    lax.fori_loop(0, NPASS, pass_body, 0)


def _sc_relation(q_tab, k_tab, v_tab, si_s, di_s, starts):
    mesh = plsc.VectorSubcoreMesh(core_axis_name="c", subcore_axis_name="s")
    kfn = pl.kernel(
        _sc_body, mesh=mesh,
        compiler_params=pltpu.CompilerParams(needs_layout_passes=False),
        out_type=[
            jax.ShapeDtypeStruct((NPAD * D,), jnp.float32),
            jax.ShapeDtypeStruct((NPAD * 16,), jnp.float32),
        ],
        scratch_types=[
            pltpu.VMEM((144,), jnp.int32),
            pltpu.VMEM((16,), jnp.int32),
            pltpu.VMEM((16,), jnp.int32),
            pltpu.VMEM((16, D), jnp.float32),
            pltpu.VMEM((16, D), jnp.float32),
            pltpu.VMEM((16, D), jnp.float32),
            pltpu.VMEM((16 * D,), jnp.float32),
            pltpu.VMEM((16 * D,), jnp.float32),
            pltpu.VMEM((16 * D,), jnp.float32),
            pltpu.VMEM((R * D,), jnp.float32),
            pltpu.VMEM((R * 16,), jnp.float32),
            pltpu.SemaphoreType.DMA,
            pltpu.SemaphoreType.DMA,
            pltpu.SemaphoreType.DMA,
        ],
    )
    num, den = kfn(q_tab, k_tab, v_tab, si_s, di_s, starts)
    return num.reshape(NPAD, D), den.reshape(NPAD, 16)


def _epi_body(x_ref, s_ref, waf_ref, baf_ref, war_ref, bar_ref, sg_ref,
              nf0, df0, nf1, df1, nf2, df2,
              nr0, dr0, nr1, dr1, nr2, dr2,
              o_ref):
    s_mat = s_ref[...]
    accf = nf0[...] / (jnp.dot(df0[...], s_mat,
                               preferred_element_type=jnp.float32) + 1e-16)
    accf = accf + nf1[...] / (jnp.dot(df1[...], s_mat,
                                      preferred_element_type=jnp.float32) + 1e-16)
    accf = accf + nf2[...] / (jnp.dot(df2[...], s_mat,
                                      preferred_element_type=jnp.float32) + 1e-16)
    accr = nr0[...] / (jnp.dot(dr0[...], s_mat,
                               preferred_element_type=jnp.float32) + 1e-16)
    accr = accr + nr1[...] / (jnp.dot(dr1[...], s_mat,
                                      preferred_element_type=jnp.float32) + 1e-16)
    accr = accr + nr2[...] / (jnp.dot(dr2[...], s_mat,
                                      preferred_element_type=jnp.float32) + 1e-16)
    of = jnp.dot(jax.nn.gelu(accf), waf_ref[...],
                 preferred_element_type=jnp.float32) + baf_ref[...]
    orv = jnp.dot(jax.nn.gelu(accr), war_ref[...],
                  preferred_element_type=jnp.float32) + bar_ref[...]
    sgf = sg_ref[0, 0]
    sgr = sg_ref[0, 1]
    x = x_ref[...]
    resf = sgf * of + (1.0 - sgf) * x
    resr = sgr * orv + (1.0 - sgr) * x
    o_ref[...] = 0.5 * (resf + resr)


def _epilogue(x, waf, baf, war, bar, sgf, sgr,
              numf, denf, numr, denr, blk=2000):
    zn = jnp.zeros((N, D), jnp.float32)
    zd = jnp.zeros((N, 16), jnp.float32)
    while len(numf) < 3:
        numf = numf + [zn]
        denf = denf + [zd]
    while len(numr) < 3:
        numr = numr + [zn]
        denr = denr + [zd]
    s_mat = jnp.zeros((16, D), jnp.float32)
    heads = jnp.arange(D) // DH
    s_mat = s_mat.at[heads, jnp.arange(D)].set(1.0)
    sg = jnp.stack([sgf, sgr]).reshape(1, 2)

    grid = N // blk
    bspec_d = pl.BlockSpec((blk, D), lambda i: (i, 0))
    bspec_16 = pl.BlockSpec((blk, 16), lambda i: (i, 0))
    full = lambda shape: pl.BlockSpec(shape, lambda i: (0, 0))
    return pl.pallas_call(
        _epi_body,
        grid=(grid,),
        in_specs=[
            bspec_d, full((16, D)), full((D, D)), full((1, D)),
            full((D, D)), full((1, D)), full((1, 2)),
            bspec_d, bspec_16, bspec_d, bspec_16, bspec_d, bspec_16,
            bspec_d, bspec_16, bspec_d, bspec_16, bspec_d, bspec_16,
        ],
        out_specs=bspec_d,
        out_shape=jax.ShapeDtypeStruct((N, D), jnp.float32),
    )(x, s_mat, waf, baf.reshape(1, D), war, bar.reshape(1, D), sg,
      numf[0], denf[0], numf[1], denf[1], numf[2], denf[2],
      numr[0], denr[0], numr[1], denr[1], numr[2], denr[2])


def _compose_k(wk, bk, rel_a, p):
    scale = (p / jnp.sqrt(float(DH))).astype(jnp.float32)
    w = jnp.einsum('dhe,hef->dhf', wk.reshape(D, H, DH), rel_a)
    w = w * scale[None, :, None]
    b = jnp.einsum('he,hef->hf', bk.reshape(H, DH), rel_a) * scale[:, None]
    return w.reshape(D, D), b.reshape(D)


def _compose_v(wv, bv, rel_m):
    w = jnp.einsum('dhe,hef->dhf', wv.reshape(D, H, DH), rel_m)
    b = jnp.einsum('he,hef->hf', bv.reshape(H, DH), rel_m)
    return w.reshape(D, D), b.reshape(D)


def kernel(x_inst, x_data, ei_control, ei_input, ei_output, ei_call, params):
    x = {'inst': x_inst, 'data': x_data}
    rels = [
        ('fwd', 'control', 'inst', 'inst', ei_control[0], ei_control[1]),
        ('fwd', 'input', 'data', 'inst', ei_input[0], ei_input[1]),
        ('fwd', 'output', 'inst', 'data', ei_output[0], ei_output[1]),
        ('fwd', 'call', 'inst', 'inst', ei_call[0], ei_call[1]),
        ('rev', 'be_control', 'inst', 'inst', ei_control[1], ei_control[0]),
        ('rev', 'be_input', 'inst', 'data', ei_input[1], ei_input[0]),
        ('rev', 'be_output', 'data', 'inst', ei_output[1], ei_output[0]),
        ('rev', 'be_call', 'inst', 'inst', ei_call[1], ei_call[0]),
    ]

    segs = {'inst': [], 'data': []}
    for dirn in ('fwd', 'rev'):
        p = params[dirn]
        for t in ('inst', 'data'):
            segs[t].append((('q', dirn, t), p['Wq'][t], p['bq'][t]))
    for (dirn, r, s, d, si, di) in rels:
        p = params[dirn]
        wk, bk = _compose_k(p['Wk'][s], p['bk'][s], p['rel'][r]['a'],
                            p['rel'][r]['p'])
        wv, bv = _compose_v(p['Wv'][s], p['bv'][s], p['rel'][r]['m'])
        segs[s].append((('k', dirn, r), wk, bk))
        segs[s].append((('v', dirn, r), wv, bv))

    tabs = {}
    for t in ('inst', 'data'):
        wbig = jnp.concatenate([w for (_, w, _) in segs[t]], axis=1)
        bbig = jnp.concatenate([b for (_, _, b) in segs[t]], axis=0)
        y = _project(x[t], wbig, bbig)
        for i, (tag, _, _) in enumerate(segs[t]):
            tabs[tag] = y[:, i * D:(i + 1) * D]

    bounds = jnp.arange(NW * NPASS + 1, dtype=jnp.int32) * R
    nums = {}
    dens = {}
    for (dirn, r, s, d, si, di) in rels:
        order = jnp.argsort(di)
        si_s = si[order].astype(jnp.int32)
        di_s = di[order].astype(jnp.int32)
        starts = jnp.searchsorted(di_s, bounds, side='left').astype(jnp.int32)
        starts = jnp.concatenate(
            [starts, jnp.full((144 - NW * NPASS - 1,), E, jnp.int32)])
        num, den = _sc_relation(tabs[('q', dirn, d)], tabs[('k', dirn, r)],
                                tabs[('v', dirn, r)], si_s, di_s, starts)
        nums[(dirn, r)] = num[:N]
        dens[(dirn, r)] = den[:N]

    by_dst = {
        ('fwd', 'inst'): ['control', 'input', 'call'],
        ('fwd', 'data'): ['output'],
        ('rev', 'inst'): ['be_control', 'be_output', 'be_call'],
        ('rev', 'data'): ['be_input'],
    }
    outs = {}
    for t in ('inst', 'data'):
        numf = [nums[('fwd', r)] for r in by_dst[('fwd', t)]]
        denf = [dens[('fwd', r)] for r in by_dst[('fwd', t)]]
        numr = [nums[('rev', r)] for r in by_dst[('rev', t)]]
        denr = [dens[('rev', r)] for r in by_dst[('rev', t)]]
        pf, pr = params['fwd'], params['rev']
        sgf = jax.nn.sigmoid(pf['skip'][t])
        sgr = jax.nn.sigmoid(pr['skip'][t])
        outs[t] = _epilogue(x[t], pf['Wa'][t], pf['ba'][t],
                            pr['Wa'][t], pr['ba'][t], sgf, sgr,
                            numf, denf, numr, denr)
    return (outs['inst'], outs['data'])
